# Initial kernel scaffold; baseline (speedup 1.0000x reference)
import jax
import jax.numpy as jnp
from jax.experimental import pallas as pl


def kernel(x, edge_index, W1, b1, W2, b2):
    raise NotImplementedError("write your pallas kernel here")


# speedup vs baseline: 2.8451x; 2.8451x over previous
import functools

import jax
import jax.numpy as jnp
from jax import lax
from jax.experimental import pallas as pl
from jax.experimental.pallas import tpu as pltpu
from jax.experimental.pallas import tpu_sc as plsc

N_NODES = 10000
N_EDGES = 320000
D = 128

NP = 10240
EP = 327680
N_TILES = 32
EPT = EP // N_TILES
CHUNK = 128
NCH = EPT // CHUNK
RPS = NP // 16
TB = 256

_mesh = plsc.VectorSubcoreMesh(core_axis_name="c", subcore_axis_name="s")
_sc_params = pltpu.CompilerParams(needs_layout_passes=False)


@functools.partial(
    pl.kernel,
    out_type=(
        jax.ShapeDtypeStruct((N_TILES, NP), jnp.float32),
        jax.ShapeDtypeStruct((N_TILES, NP), jnp.float32),
    ),
    mesh=_mesh,
    scratch_types=[
        pltpu.VMEM((EPT,), jnp.int32),
        pltpu.VMEM((NP,), jnp.float32),
        pltpu.VMEM((NP,), jnp.float32),
    ],
    compiler_params=_sc_params,
)
def _deg_kernel(src_hbm, dst_hbm, z1_hbm, outs_hbm, outd_hbm, idx_v, hs_v, hd_v):
    w = lax.axis_index("s") * 2 + lax.axis_index("c")
    pltpu.sync_copy(z1_hbm, hs_v)
    pltpu.sync_copy(z1_hbm, hd_v)
    ones = jnp.ones((16,), jnp.float32)

    def accum(idx_hbm, hist):
        pltpu.sync_copy(idx_hbm.at[w], idx_v)

        def body(g, carry):
            idx16 = idx_v[pl.ds(g * 16, 16)]
            plsc.addupdate_scatter(hist, [idx16], ones)
            return carry

        lax.fori_loop(0, EPT // 16, body, 0)

    accum(src_hbm, hs_v)
    accum(dst_hbm, hd_v)
    pltpu.sync_copy(hs_v, outs_hbm.at[w])
    pltpu.sync_copy(hd_v, outd_hbm.at[w])


@functools.partial(
    pl.kernel,
    out_type=jax.ShapeDtypeStruct((2, NP, D), jnp.float32),
    mesh=_mesh,
    scratch_types=[
        pltpu.VMEM((NCH, CHUNK), jnp.int32),
        pltpu.VMEM((NCH, CHUNK), jnp.int32),
        pltpu.VMEM((CHUNK, D), jnp.float32),
        pltpu.VMEM_SHARED((NP, D), jnp.float32),
        pltpu.SemaphoreType.DMA,
    ],
    compiler_params=_sc_params,
)
def _prop_kernel(t_hbm, src_hbm, dst_hbm, z2_hbm, out_hbm, si_v, di_v, rows_v,
                 acc_sh, sem):
    c = lax.axis_index("c")
    s = lax.axis_index("s")
    w = s * 2 + c
    pltpu.sync_copy(z2_hbm, acc_sh.at[pl.ds(s * RPS, RPS)])
    pltpu.sync_copy(src_hbm.at[w], si_v)
    pltpu.sync_copy(dst_hbm.at[w], di_v)
    plsc.subcore_barrier()

    def body(j, carry):
        pltpu.async_copy(t_hbm.at[si_v.at[j]], rows_v, sem).wait()
        pltpu.sync_copy(rows_v, acc_sh.at[di_v.at[j]], add=True)
        return carry

    lax.fori_loop(0, NCH, body, 0)
    plsc.subcore_barrier()
    pltpu.sync_copy(acc_sh.at[pl.ds(s * RPS, RPS)],
                    out_hbm.at[c, pl.ds(s * RPS, RPS)])


def _norm_prescale_body(ps_ref, pd_ref, x_ref, t0_ref, ns_ref, nd_ref):
    degs = jnp.sum(ps_ref[...], axis=0)
    degd = jnp.sum(pd_ref[...], axis=0)
    nsv = lax.rsqrt(jnp.maximum(degs, 1.0))
    ndv = lax.rsqrt(jnp.maximum(degd, 1.0))
    t0_ref[...] = x_ref[...] * nsv[:, None]
    ns_ref[...] = nsv[:, None]
    nd_ref[...] = ndv[:, None]


_norm_prescale = pl.pallas_call(
    _norm_prescale_body,
    grid=(NP // TB,),
    in_specs=[
        pl.BlockSpec((N_TILES, TB), lambda i: (0, i)),
        pl.BlockSpec((N_TILES, TB), lambda i: (0, i)),
        pl.BlockSpec((TB, D), lambda i: (i, 0)),
    ],
    out_specs=[
        pl.BlockSpec((TB, D), lambda i: (i, 0)),
        pl.BlockSpec((TB, 1), lambda i: (i, 0)),
        pl.BlockSpec((TB, 1), lambda i: (i, 0)),
    ],
    out_shape=[
        jax.ShapeDtypeStruct((NP, D), jnp.float32),
        jax.ShapeDtypeStruct((NP, 1), jnp.float32),
        jax.ShapeDtypeStruct((NP, 1), jnp.float32),
    ],
)


def _mid_body(acc_ref, nd_ref, ns_ref, w_ref, b_ref, t1_ref):
    a = (acc_ref[0] + acc_ref[1]) * nd_ref[...]
    h = jnp.dot(a, w_ref[...], preferred_element_type=jnp.float32) + b_ref[...]
    t1_ref[...] = h * ns_ref[...]


_mid_layer = pl.pallas_call(
    _mid_body,
    grid=(NP // TB,),
    in_specs=[
        pl.BlockSpec((2, TB, D), lambda i: (0, i, 0)),
        pl.BlockSpec((TB, 1), lambda i: (i, 0)),
        pl.BlockSpec((TB, 1), lambda i: (i, 0)),
        pl.BlockSpec((D, D), lambda i: (0, 0)),
        pl.BlockSpec((1, D), lambda i: (0, 0)),
    ],
    out_specs=pl.BlockSpec((TB, D), lambda i: (i, 0)),
    out_shape=jax.ShapeDtypeStruct((NP, D), jnp.float32),
)


def _out_body(acc_ref, nd_ref, w_ref, b_ref, o_ref):
    a = (acc_ref[0] + acc_ref[1]) * nd_ref[...]
    o_ref[...] = jnp.dot(a, w_ref[...], preferred_element_type=jnp.float32) + b_ref[...]


_out_layer = pl.pallas_call(
    _out_body,
    grid=(NP // TB,),
    in_specs=[
        pl.BlockSpec((2, TB, D), lambda i: (0, i, 0)),
        pl.BlockSpec((TB, 1), lambda i: (i, 0)),
        pl.BlockSpec((D, D), lambda i: (0, 0)),
        pl.BlockSpec((1, D), lambda i: (0, 0)),
    ],
    out_specs=pl.BlockSpec((TB, D), lambda i: (i, 0)),
    out_shape=jax.ShapeDtypeStruct((NP, D), jnp.float32),
)


def kernel(x, edge_index, W1, b1, W2, b2):
    src = edge_index[0].astype(jnp.int32)
    dst = edge_index[1].astype(jnp.int32)
    padi = jnp.full((EP - N_EDGES,), NP - 1, jnp.int32)
    src_p = jnp.concatenate([src, padi])
    dst_p = jnp.concatenate([dst, padi])
    src2 = src_p.reshape(N_TILES, EPT)
    dst2 = dst_p.reshape(N_TILES, EPT)
    src3 = src_p.reshape(N_TILES, NCH, CHUNK)
    dst3 = dst_p.reshape(N_TILES, NCH, CHUNK)
    z1 = jnp.zeros((NP,), jnp.float32)
    z2 = jnp.zeros((RPS, D), jnp.float32)
    x_p = jnp.pad(x, ((0, NP - N_NODES), (0, 0)))

    hs, hd = _deg_kernel(src2, dst2, z1)
    t0, ns, nd = _norm_prescale(hs, hd, x_p)
    acc1 = _prop_kernel(t0, src3, dst3, z2)
    t1 = _mid_layer(acc1, nd, ns, W1, b1.reshape(1, D))
    acc2 = _prop_kernel(t1, src3, dst3, z2)
    out = _out_layer(acc2, nd, W2, b2.reshape(1, D))
    return out[:N_NODES]


# speedup vs baseline: 4.7345x; 1.6641x over previous
import functools

import jax
import jax.numpy as jnp
from jax import lax
from jax.experimental import pallas as pl
from jax.experimental.pallas import tpu as pltpu
from jax.experimental.pallas import tpu_sc as plsc

N_NODES = 10000
N_EDGES = 320000
D = 128
DH = 64

NP = 10240
EP = 327680
N_TILES = 32
EPT1 = EP // N_TILES
EPT = EP // 16
CHUNK = 128
NCH = EPT // CHUNK
RPS = NP // 16
NBUF = 4
TB = 256

_mesh = plsc.VectorSubcoreMesh(core_axis_name="c", subcore_axis_name="s")
_sc_params = pltpu.CompilerParams(
    needs_layout_passes=False, use_tc_tiling_on_sc=False
)


@functools.partial(
    pl.kernel,
    out_type=(
        jax.ShapeDtypeStruct((N_TILES, NP), jnp.float32),
        jax.ShapeDtypeStruct((N_TILES, NP), jnp.float32),
    ),
    mesh=_mesh,
    scratch_types=[
        pltpu.VMEM((EPT1,), jnp.int32),
        pltpu.VMEM((NP,), jnp.float32),
        pltpu.VMEM((NP,), jnp.float32),
    ],
    compiler_params=_sc_params,
)
def _deg_kernel(src_hbm, dst_hbm, z1_hbm, outs_hbm, outd_hbm, idx_v, hs_v, hd_v):
    w = lax.axis_index("s") * 2 + lax.axis_index("c")
    pltpu.sync_copy(z1_hbm, hs_v)
    pltpu.sync_copy(z1_hbm, hd_v)
    ones = jnp.ones((16,), jnp.float32)

    def accum(idx_hbm, hist):
        pltpu.sync_copy(idx_hbm.at[w], idx_v)

        def body(g, carry):
            idx16 = idx_v[pl.ds(g * 16, 16)]
            plsc.addupdate_scatter(hist, [idx16], ones)
            return carry

        lax.fori_loop(0, EPT1 // 16, body, 0)

    accum(src_hbm, hs_v)
    accum(dst_hbm, hd_v)
    pltpu.sync_copy(hs_v, outs_hbm.at[w])
    pltpu.sync_copy(hd_v, outd_hbm.at[w])


@functools.partial(
    pl.kernel,
    out_type=jax.ShapeDtypeStruct((2, NP, DH), jnp.float32),
    mesh=_mesh,
    scratch_types=[
        pltpu.VMEM((NCH, CHUNK), jnp.int32),
        pltpu.VMEM((NCH, CHUNK), jnp.int32),
        pltpu.VMEM((NBUF, CHUNK, DH), jnp.float32),
        pltpu.VMEM_SHARED((NP, DH), jnp.float32),
        pltpu.SemaphoreType.DMA,
        pltpu.SemaphoreType.DMA,
        pltpu.SemaphoreType.DMA,
        pltpu.SemaphoreType.DMA,
    ],
    compiler_params=_sc_params,
)
def _prop_kernel(t_hbm, src_hbm, dst_hbm, z2_hbm, out_hbm, si_v, di_v, rows_v,
                 acc_sh, sem0, sem1, sem2, sem3):
    sems = [sem0, sem1, sem2, sem3]
    c = lax.axis_index("c")
    s = lax.axis_index("s")
    th = t_hbm.at[c]
    pltpu.sync_copy(z2_hbm, acc_sh.at[pl.ds(s * RPS, RPS)])
    pltpu.sync_copy(src_hbm.at[s], si_v)
    pltpu.sync_copy(dst_hbm.at[s], di_v)
    plsc.subcore_barrier()

    for b in range(NBUF):
        pltpu.async_copy(th.at[si_v.at[b]], rows_v.at[b], sems[b])

    def body(j4, carry):
        for b in range(NBUF):
            j = j4 * NBUF + b
            pltpu.make_async_copy(th.at[si_v.at[j]], rows_v.at[b],
                                  sems[b]).wait()
            pltpu.sync_copy(rows_v.at[b], acc_sh.at[di_v.at[j]], add=True)
            pltpu.async_copy(th.at[si_v.at[j + NBUF]], rows_v.at[b], sems[b])
        return carry

    lax.fori_loop(0, NCH // NBUF - 1, body, 0)
    for b in range(NBUF):
        j = NCH - NBUF + b
        pltpu.make_async_copy(th.at[si_v.at[j]], rows_v.at[b], sems[b]).wait()
        pltpu.sync_copy(rows_v.at[b], acc_sh.at[di_v.at[j]], add=True)

    plsc.subcore_barrier()
    pltpu.sync_copy(acc_sh.at[pl.ds(s * RPS, RPS)],
                    out_hbm.at[c, pl.ds(s * RPS, RPS)])


def _norm_prescale_body(ps_ref, pd_ref, x_ref, t0_ref, ns_ref, nd_ref):
    degs = jnp.sum(ps_ref[...], axis=0)
    degd = jnp.sum(pd_ref[...], axis=0)
    nsv = lax.rsqrt(jnp.maximum(degs, 1.0))
    ndv = lax.rsqrt(jnp.maximum(degd, 1.0))
    t0 = x_ref[...] * nsv[:, None]
    t0_ref[...] = jnp.stack([t0[:, :DH], t0[:, DH:]], axis=0)
    ns_ref[...] = nsv[:, None]
    nd_ref[...] = ndv[:, None]


_norm_prescale = pl.pallas_call(
    _norm_prescale_body,
    grid=(NP // TB,),
    in_specs=[
        pl.BlockSpec((N_TILES, TB), lambda i: (0, i)),
        pl.BlockSpec((N_TILES, TB), lambda i: (0, i)),
        pl.BlockSpec((TB, D), lambda i: (i, 0)),
    ],
    out_specs=[
        pl.BlockSpec((2, TB, DH), lambda i: (0, i, 0)),
        pl.BlockSpec((TB, 1), lambda i: (i, 0)),
        pl.BlockSpec((TB, 1), lambda i: (i, 0)),
    ],
    out_shape=[
        jax.ShapeDtypeStruct((2, NP, DH), jnp.float32),
        jax.ShapeDtypeStruct((NP, 1), jnp.float32),
        jax.ShapeDtypeStruct((NP, 1), jnp.float32),
    ],
)


def _mid_body(acc_ref, nd_ref, ns_ref, w_ref, b_ref, t1_ref):
    a = jnp.concatenate([acc_ref[0], acc_ref[1]], axis=1) * nd_ref[...]
    h = jnp.dot(a, w_ref[...], preferred_element_type=jnp.float32) + b_ref[...]
    t1 = h * ns_ref[...]
    t1_ref[...] = jnp.stack([t1[:, :DH], t1[:, DH:]], axis=0)


_mid_layer = pl.pallas_call(
    _mid_body,
    grid=(NP // TB,),
    in_specs=[
        pl.BlockSpec((2, TB, DH), lambda i: (0, i, 0)),
        pl.BlockSpec((TB, 1), lambda i: (i, 0)),
        pl.BlockSpec((TB, 1), lambda i: (i, 0)),
        pl.BlockSpec((D, D), lambda i: (0, 0)),
        pl.BlockSpec((1, D), lambda i: (0, 0)),
    ],
    out_specs=pl.BlockSpec((2, TB, DH), lambda i: (0, i, 0)),
    out_shape=jax.ShapeDtypeStruct((2, NP, DH), jnp.float32),
)


def _out_body(acc_ref, nd_ref, w_ref, b_ref, o_ref):
    a = jnp.concatenate([acc_ref[0], acc_ref[1]], axis=1) * nd_ref[...]
    o_ref[...] = jnp.dot(a, w_ref[...], preferred_element_type=jnp.float32) + b_ref[...]


_out_layer = pl.pallas_call(
    _out_body,
    grid=(NP // TB,),
    in_specs=[
        pl.BlockSpec((2, TB, DH), lambda i: (0, i, 0)),
        pl.BlockSpec((TB, 1), lambda i: (i, 0)),
        pl.BlockSpec((D, D), lambda i: (0, 0)),
        pl.BlockSpec((1, D), lambda i: (0, 0)),
    ],
    out_specs=pl.BlockSpec((TB, D), lambda i: (i, 0)),
    out_shape=jax.ShapeDtypeStruct((NP, D), jnp.float32),
)


def kernel(x, edge_index, W1, b1, W2, b2):
    src = edge_index[0].astype(jnp.int32)
    dst = edge_index[1].astype(jnp.int32)
    padi = jnp.full((EP - N_EDGES,), NP - 1, jnp.int32)
    src_p = jnp.concatenate([src, padi])
    dst_p = jnp.concatenate([dst, padi])
    src2 = src_p.reshape(N_TILES, EPT1)
    dst2 = dst_p.reshape(N_TILES, EPT1)
    src3 = src_p.reshape(16, NCH, CHUNK)
    dst3 = dst_p.reshape(16, NCH, CHUNK)
    z1 = jnp.zeros((NP,), jnp.float32)
    z2 = jnp.zeros((RPS, DH), jnp.float32)
    x_p = jnp.pad(x, ((0, NP - N_NODES), (0, 0)))

    hs, hd = _deg_kernel(src2, dst2, z1)
    t0, ns, nd = _norm_prescale(hs, hd, x_p)
    acc1 = _prop_kernel(t0, src3, dst3, z2)
    t1 = _mid_layer(acc1, nd, ns, W1, b1.reshape(1, D))
    acc2 = _prop_kernel(t1, src3, dst3, z2)
    out = _out_layer(acc2, nd, W2, b2.reshape(1, D))
    return out[:N_NODES]
